# Initial kernel scaffold; baseline (speedup 1.0000x reference)
import jax
import jax.numpy as jnp
from jax.experimental import pallas as pl


def kernel(Z, edge_index, edge_diff, embed, msg_W1, msg_b1, msg_W2, msg_b2, filt_W, filt_b, upd_U, upd_V, upd_W1, upd_b1, upd_W2, upd_b2, ro_W1, ro_b1, ro_W2, ro_b2):
    raise NotImplementedError("write your pallas kernel here")


# speedup vs baseline: 14.2480x; 14.2480x over previous
import functools

import jax
import jax.numpy as jnp
from jax import lax
from jax.experimental import pallas as pl
from jax.experimental.pallas import tpu as pltpu
from jax.experimental.pallas import tpu_sc as plsc

N = 10000
E = 320000
F = 128
NB = 20
CUTOFF = 5.0
PI = 3.141592653589793

NC = 2
NS = 16
NW = NC * NS

CH = 128


def _silu(x):
    return x * jax.nn.sigmoid(x)


def _make_sc_gather(T, D, B):
    assert B % NW == 0
    bpw = B // NW
    full, tail = divmod(bpw, CH)
    mesh = plsc.VectorSubcoreMesh(core_axis_name="c", subcore_axis_name="s",
                                  num_cores=NC, num_subcores=NS)
    scratch = [
        pltpu.VMEM((CH,), jnp.int32),
        pltpu.VMEM((CH, D), jnp.float32),
        pltpu.SemaphoreType.DMA,
    ]
    if tail:
        scratch += [
            pltpu.VMEM((tail,), jnp.int32),
            pltpu.VMEM((tail, D), jnp.float32),
        ]

    @functools.partial(
        pl.kernel,
        out_type=jax.ShapeDtypeStruct((B, D), jnp.float32),
        mesh=mesh,
        scratch_types=scratch,
    )
    def gather(table, idx, out, idx_v, rows_v, sem, *tail_bufs):
        wid = lax.axis_index("s") * NC + lax.axis_index("c")
        base = wid * bpw

        def body(i, carry):
            off = base + i * CH
            pltpu.sync_copy(idx.at[pl.ds(off, CH)], idx_v)
            pltpu.async_copy(table.at[idx_v], rows_v, sem).wait()
            pltpu.sync_copy(rows_v, out.at[pl.ds(off, CH)])
            return carry

        lax.fori_loop(0, full, body, 0)
        if tail:
            idx_t, rows_t = tail_bufs
            off = base + full * CH
            pltpu.sync_copy(idx.at[pl.ds(off, tail)], idx_t)
            pltpu.async_copy(table.at[idx_t], rows_t, sem).wait()
            pltpu.sync_copy(rows_t, out.at[pl.ds(off, tail)])

    return gather


def _make_sc_scatter(ncomp):
    epc = E // NC
    ept = epc // NS
    full, tail = divmod(ept, CH)
    mesh = plsc.VectorSubcoreMesh(core_axis_name="c", subcore_axis_name="s",
                                  num_cores=NC, num_subcores=NS)
    scratch = [
        pltpu.VMEM((CH,), jnp.int32),
        pltpu.VMEM((CH, F), jnp.float32),
        pltpu.SemaphoreType.DMA,
        pltpu.VMEM_SHARED((N, F), jnp.float32),
    ]
    if tail:
        scratch += [
            pltpu.VMEM((tail,), jnp.int32),
            pltpu.VMEM((tail, F), jnp.float32),
        ]

    @functools.partial(
        pl.kernel,
        out_type=jax.ShapeDtypeStruct((ncomp * NC, N, F), jnp.float32),
        mesh=mesh,
        scratch_types=scratch,
    )
    def scatter(*args):
        msgs = args[:ncomp]
        dst, zeros, out, idx_v, msg_v, sem, accum = args[ncomp:ncomp + 7]
        tail_bufs = args[ncomp + 7:]
        c = lax.axis_index("c")
        s = lax.axis_index("s")
        base = c * epc + s * ept

        for comp in range(ncomp):
            msg = msgs[comp]

            @pl.when(s == 0)
            def _():
                pltpu.sync_copy(zeros, accum)

            plsc.subcore_barrier()

            def body(i, carry):
                off = base + i * CH
                pltpu.sync_copy(dst.at[pl.ds(off, CH)], idx_v)
                pltpu.sync_copy(msg.at[pl.ds(off, CH)], msg_v)
                pltpu.sync_copy(msg_v, accum.at[idx_v], add=True)
                return carry

            lax.fori_loop(0, full, body, 0)
            if tail:
                idx_t, msg_t = tail_bufs
                off = base + full * CH
                pltpu.sync_copy(dst.at[pl.ds(off, tail)], idx_t)
                pltpu.sync_copy(msg.at[pl.ds(off, tail)], msg_t)
                pltpu.sync_copy(msg_t, accum.at[idx_t], add=True)

            plsc.subcore_barrier()

            @pl.when(s == 0)
            def _():
                pltpu.sync_copy(accum, out.at[comp * NC + c])

            plsc.subcore_barrier()

    return scatter


_NBLK = 2000


def _node_dense_body(nf_ref, w1_ref, b1_ref, w2_ref, b2_ref, out_ref):
    h = _silu(nf_ref[...] @ w1_ref[...] + b1_ref[...])
    out_ref[...] = h @ w2_ref[...] + b2_ref[...]


def _node_dense(nf, w1, b1, w2, b2):
    return pl.pallas_call(
        _node_dense_body,
        grid=(N // _NBLK,),
        in_specs=[
            pl.BlockSpec((_NBLK, F), lambda i: (i, 0)),
            pl.BlockSpec((F, F), lambda i: (0, 0)),
            pl.BlockSpec((F,), lambda i: (0,)),
            pl.BlockSpec((F, 3 * F), lambda i: (0, 0)),
            pl.BlockSpec((3 * F,), lambda i: (0,)),
        ],
        out_specs=pl.BlockSpec((_NBLK, 3 * F), lambda i: (i, 0)),
        out_shape=jax.ShapeDtypeStruct((N, 3 * F), jnp.float32),
    )(nf, w1, b1, w2, b2)


_EBLK = 2000


def _edge_body_common(diff, g_sc, fw_ref, fb_ref):
    d2 = jnp.sum(diff * diff, axis=1, keepdims=True)
    d = jnp.sqrt(d2 + 1e-12)
    k = lax.broadcasted_iota(jnp.int32, (diff.shape[0], NB), 1).astype(jnp.float32) + 1.0
    rbf = jnp.sin(d * k * (PI / CUTOFF)) / d
    fcut = jnp.where(d < CUTOFF, 0.5 * (jnp.cos(d * (PI / CUTOFF)) + 1.0), 0.0)
    filt = (rbf @ fw_ref[...] + fb_ref[...]) * fcut
    fo = filt * g_sc
    gate_v = fo[:, :F]
    gate_e = fo[:, F:2 * F]
    msg_s = fo[:, 2 * F:]
    unit = diff / d
    return gate_v, gate_e, msg_s, unit


def _edge_body(ed_ref, g_ref, fw_ref, fb_ref, ms_ref, mx_ref, my_ref, mz_ref):
    diff = ed_ref[...]
    g = g_ref[...]
    gate_v, gate_e, msg_s, unit = _edge_body_common(diff, g[:, :3 * F], fw_ref, fb_ref)
    ms_ref[...] = msg_s
    mx_ref[...] = g[:, 3 * F:4 * F] * gate_v + gate_e * unit[:, 0:1]
    my_ref[...] = g[:, 4 * F:5 * F] * gate_v + gate_e * unit[:, 1:2]
    mz_ref[...] = g[:, 5 * F:6 * F] * gate_v + gate_e * unit[:, 2:3]


def _edge_body_l0(ed_ref, g_ref, fw_ref, fb_ref, ms_ref, mx_ref, my_ref, mz_ref):
    diff = ed_ref[...]
    gate_v, gate_e, msg_s, unit = _edge_body_common(diff, g_ref[...], fw_ref, fb_ref)
    ms_ref[...] = msg_s
    mx_ref[...] = gate_e * unit[:, 0:1]
    my_ref[...] = gate_e * unit[:, 1:2]
    mz_ref[...] = gate_e * unit[:, 2:3]


def _edge_messages(edge_diff, gathered, fw, fb, first_layer):
    gdim = 3 * F if first_layer else 6 * F
    body = _edge_body_l0 if first_layer else _edge_body
    out_sd = jax.ShapeDtypeStruct((E, F), jnp.float32)
    out_spec = pl.BlockSpec((_EBLK, F), lambda i: (i, 0))
    return pl.pallas_call(
        body,
        grid=(E // _EBLK,),
        in_specs=[
            pl.BlockSpec((_EBLK, 3), lambda i: (i, 0)),
            pl.BlockSpec((_EBLK, gdim), lambda i: (i, 0)),
            pl.BlockSpec((NB, 3 * F), lambda i: (0, 0)),
            pl.BlockSpec((3 * F,), lambda i: (0,)),
        ],
        out_specs=[out_spec, out_spec, out_spec, out_spec],
        out_shape=[out_sd, out_sd, out_sd, out_sd],
    )(edge_diff, gathered, fw, fb)


def _update_body(nf_ref, nvx_ref, nvy_ref, nvz_ref, seg_ref,
                 u_ref, v_ref, w1_ref, b1_ref, w2_ref, b2_ref,
                 nf_out, nvx_out, nvy_out, nvz_out):
    seg = seg_ref[...]
    f = nf_ref[...] + seg[0] + seg[1]
    vx = nvx_ref[...] + seg[2] + seg[3]
    vy = nvy_ref[...] + seg[4] + seg[5]
    vz = nvz_ref[...] + seg[6] + seg[7]
    U = u_ref[...]
    V = v_ref[...]
    uvx = vx @ U
    uvy = vy @ U
    uvz = vz @ U
    vvx = vx @ V
    vvy = vy @ V
    vvz = vz @ V
    vn = jnp.sqrt(vvx * vvx + vvy * vvy + vvz * vvz + 1e-12)
    w1 = w1_ref[...]
    h = _silu(f @ w1[:F, :] + vn @ w1[F:, :] + b1_ref[...])
    a = h @ w2_ref[...] + b2_ref[...]
    a_vv = a[:, :F]
    a_sv = a[:, F:2 * F]
    a_ss = a[:, 2 * F:]
    nf_out[...] = f + (uvx * vvx + uvy * vvy + uvz * vvz) * a_sv + a_ss
    nvx_out[...] = vx + a_vv * uvx
    nvy_out[...] = vy + a_vv * uvy
    nvz_out[...] = vz + a_vv * uvz


_UBLK = 1000


def _update(nf, nv3, seg, U, V, w1, b1, w2, b2):
    nspec = pl.BlockSpec((_UBLK, F), lambda i: (i, 0))
    out_sd = jax.ShapeDtypeStruct((N, F), jnp.float32)
    seg_spec = pl.BlockSpec((4 * NC, _UBLK, F), lambda i: (0, i, 0))
    return pl.pallas_call(
        _update_body,
        grid=(N // _UBLK,),
        in_specs=[nspec, nspec, nspec, nspec, seg_spec] + [
            pl.BlockSpec((F, F), lambda i: (0, 0)),
            pl.BlockSpec((F, F), lambda i: (0, 0)),
            pl.BlockSpec((2 * F, F), lambda i: (0, 0)),
            pl.BlockSpec((F,), lambda i: (0,)),
            pl.BlockSpec((F, 3 * F), lambda i: (0, 0)),
            pl.BlockSpec((3 * F,), lambda i: (0,)),
        ],
        out_specs=[nspec, nspec, nspec, nspec],
        out_shape=[out_sd, out_sd, out_sd, out_sd],
    )(nf, nv3[0], nv3[1], nv3[2], seg, U, V, w1, b1, w2, b2)


def _readout_body(nf_ref, w1_ref, b1_ref, w2_ref, b2_ref, out_ref):
    h = _silu(nf_ref[...] @ w1_ref[...] + b1_ref[...])
    out_ref[...] = h @ w2_ref[...] + b2_ref[...]


def _readout(node_feat, ro_W1, ro_b1, ro_W2, ro_b2):
    out = pl.pallas_call(
        _readout_body,
        grid=(N // _NBLK,),
        in_specs=[
            pl.BlockSpec((_NBLK, F), lambda i: (i, 0)),
            pl.BlockSpec((F, F), lambda i: (0, 0)),
            pl.BlockSpec((F,), lambda i: (0,)),
            pl.BlockSpec((F, 1), lambda i: (0, 0)),
            pl.BlockSpec((1,), lambda i: (0,)),
        ],
        out_specs=pl.BlockSpec((_NBLK, 1), lambda i: (i, 0)),
        out_shape=jax.ShapeDtypeStruct((N, 1), jnp.float32),
    )(node_feat, ro_W1, ro_b1, ro_W2, ro_b2)
    return out[:, 0]


_N_PAD = 10240
_sc_cache = {}


def _embed_gather(table, idx):
    if "embed" not in _sc_cache:
        _sc_cache["embed"] = _make_sc_gather(119, F, _N_PAD)
    return _sc_cache["embed"](table, idx)


def _table_gather_l0(table, idx):
    if "g0" not in _sc_cache:
        _sc_cache["g0"] = _make_sc_gather(N, 3 * F, E)
    return _sc_cache["g0"](table, idx)


def _table_gather(table, idx):
    if "g" not in _sc_cache:
        _sc_cache["g"] = _make_sc_gather(N, 6 * F, E)
    return _sc_cache["g"](table, idx)


def _seg_scatter(*args):
    if "s" not in _sc_cache:
        _sc_cache["s"] = _make_sc_scatter(4)
    return _sc_cache["s"](*args)


def kernel(Z, edge_index, edge_diff, embed, msg_W1, msg_b1, msg_W2, msg_b2,
           filt_W, filt_b, upd_U, upd_V, upd_W1, upd_b1, upd_W2, upd_b2,
           ro_W1, ro_b1, ro_W2, ro_b2):
    src = edge_index[:, 1].astype(jnp.int32)
    dst = edge_index[:, 0].astype(jnp.int32)
    z_pad = jnp.zeros((_N_PAD,), jnp.int32).at[:N].set(Z.astype(jnp.int32))

    node_feat = _embed_gather(embed, z_pad)[:N]
    nv3 = None

    zeros_nf = jnp.zeros((N, F), jnp.float32)

    for l in range(3):
        sc = _node_dense(node_feat, msg_W1[l], msg_b1[l], msg_W2[l], msg_b2[l])
        if l == 0:
            gathered = _table_gather_l0(sc, src)
        else:
            table = jnp.concatenate([sc, nv3[0], nv3[1], nv3[2]], axis=1)
            gathered = _table_gather(table, src)
        ms, mx, my, mz = _edge_messages(edge_diff, gathered, filt_W[l],
                                        filt_b[l], l == 0)
        seg = _seg_scatter(ms, mx, my, mz, dst, zeros_nf)
        if l == 0:
            nv_in = (zeros_nf, zeros_nf, zeros_nf)
        else:
            nv_in = nv3
        node_feat, nvx, nvy, nvz = _update(
            node_feat, nv_in, seg, upd_U[l], upd_V[l],
            upd_W1[l], upd_b1[l], upd_W2[l], upd_b2[l])
        nv3 = (nvx, nvy, nvz)

    return _readout(node_feat, ro_W1, ro_b1, ro_W2, ro_b2)


# speedup vs baseline: 17.1071x; 1.2007x over previous
import functools

import jax
import jax.numpy as jnp
from jax import lax
from jax.experimental import pallas as pl
from jax.experimental.pallas import tpu as pltpu
from jax.experimental.pallas import tpu_sc as plsc

N = 10000
E = 320000
F = 128
NB = 20
CUTOFF = 5.0
PI = 3.141592653589793

NC = 2
NS = 16
NW = NC * NS

CH = 128


def _silu(x):
    return x * jax.nn.sigmoid(x)


def _make_sc_gather(T, D, B):
    assert B % NW == 0
    bpw = B // NW
    ch = 64 if D > 3 * F else CH
    full, tail = divmod(bpw, ch)
    assert full % 2 == 0
    pairs = full // 2
    mesh = plsc.VectorSubcoreMesh(core_axis_name="c", subcore_axis_name="s",
                                  num_cores=NC, num_subcores=NS)
    scratch = [
        pltpu.VMEM((bpw,), jnp.int32),
        pltpu.VMEM((ch, D), jnp.float32),
        pltpu.VMEM((ch, D), jnp.float32),
        pltpu.SemaphoreType.DMA,
        pltpu.SemaphoreType.DMA,
        pltpu.SemaphoreType.DMA,
        pltpu.SemaphoreType.DMA,
    ]
    if tail:
        scratch += [
            pltpu.VMEM((tail,), jnp.int32),
            pltpu.VMEM((tail, D), jnp.float32),
        ]

    @functools.partial(
        pl.kernel,
        out_type=jax.ShapeDtypeStruct((B, D), jnp.float32),
        mesh=mesh,
        scratch_types=scratch,
    )
    def gather(table, idx, out, idxb, r0, r1, gs0, gs1, os0, os1, *tail_bufs):
        wid = lax.axis_index("s") * NC + lax.axis_index("c")
        base = pl.multiple_of(wid * bpw, 8)
        pltpu.sync_copy(idx.at[pl.ds(base, bpw)], idxb)
        rows = (r0, r1)
        gsems = (gs0, gs1)
        osems = (os0, os1)
        if full:
            pltpu.async_copy(table.at[idxb.at[pl.ds(0, ch)]], r0, gs0)
            pltpu.async_copy(table.at[idxb.at[pl.ds(ch, ch)]], r1, gs1)

            def body(j, carry):
                for s in range(2):
                    i = 2 * j + s
                    ioff = pl.multiple_of(i * ch, 8)
                    pltpu.make_async_copy(
                        table.at[idxb.at[pl.ds(ioff, ch)]], rows[s],
                        gsems[s]).wait()
                    wb = pltpu.async_copy(
                        rows[s],
                        out.at[pl.ds(pl.multiple_of(base + i * ch, 8), ch)],
                        osems[s])
                    wb.wait()

                    @pl.when(j < pairs - 1)
                    def _():
                        ioff2 = pl.multiple_of((i + 2) * ch, 8)
                        pltpu.async_copy(
                            table.at[idxb.at[pl.ds(ioff2, ch)]],
                            rows[s], gsems[s])
                return carry

            lax.fori_loop(0, pairs, body, 0)
        if tail:
            idx_t, rows_t = tail_bufs
            off = pl.multiple_of(base + full * ch, 8)
            pltpu.sync_copy(idx.at[pl.ds(off, tail)], idx_t)
            pltpu.async_copy(table.at[idx_t], rows_t, gs0).wait()
            pltpu.sync_copy(rows_t, out.at[pl.ds(off, tail)])

    return gather


def _make_sc_scatter(ncomp):
    ept = E // NW
    full, tail = divmod(ept, CH)
    assert full % 2 == 0
    pairs = full // 2
    mesh = plsc.VectorSubcoreMesh(core_axis_name="c", subcore_axis_name="s",
                                  num_cores=NC, num_subcores=NS)
    scratch = [
        pltpu.VMEM((CH,), jnp.int32),
        pltpu.VMEM((CH,), jnp.int32),
        pltpu.VMEM((CH, F), jnp.float32),
        pltpu.VMEM((CH, F), jnp.float32),
        pltpu.SemaphoreType.DMA,
        pltpu.SemaphoreType.DMA,
        pltpu.SemaphoreType.DMA,
        pltpu.SemaphoreType.DMA,
        pltpu.VMEM_SHARED((N, F), jnp.float32),
        pltpu.VMEM((tail,), jnp.int32),
        pltpu.VMEM((tail, F), jnp.float32),
    ]

    @functools.partial(
        pl.kernel,
        out_type=jax.ShapeDtypeStruct((ncomp * NC, N, F), jnp.float32),
        mesh=mesh,
        scratch_types=scratch,
    )
    def scatter(*args):
        msgs = args[:ncomp]
        (dst, zeros, out, i0, i1, m0, m1, is0, is1, ls0, ls1, accum,
         idx_t, msg_t) = args[ncomp:]
        c = lax.axis_index("c")
        s = lax.axis_index("s")
        w = c * NS + s
        base = pl.multiple_of(w * ept, 8)
        ibufs = (i0, i1)
        mbufs = (m0, m1)
        isems = (is0, is1)
        msems = (ls0, ls1)

        for comp in range(ncomp):
            msg = msgs[comp]

            @pl.when(s == 0)
            def _():
                pltpu.sync_copy(zeros, accum)

            plsc.subcore_barrier()

            for sl in range(2):
                off = pl.multiple_of(base + sl * CH, 8)
                pltpu.async_copy(dst.at[pl.ds(off, CH)], ibufs[sl], isems[sl])
                pltpu.async_copy(msg.at[pl.ds(off, CH)], mbufs[sl], msems[sl])

            def body(j, carry):
                for sl in range(2):
                    i = 2 * j + sl
                    off = pl.multiple_of(base + i * CH, 8)
                    pltpu.make_async_copy(dst.at[pl.ds(off, CH)], ibufs[sl],
                                          isems[sl]).wait()
                    pltpu.make_async_copy(msg.at[pl.ds(off, CH)], mbufs[sl],
                                          msems[sl]).wait()
                    pltpu.sync_copy(mbufs[sl], accum.at[ibufs[sl]], add=True)

                    @pl.when(j < pairs - 1)
                    def _():
                        off2 = pl.multiple_of(base + (i + 2) * CH, 8)
                        pltpu.async_copy(dst.at[pl.ds(off2, CH)], ibufs[sl],
                                         isems[sl])
                        pltpu.async_copy(msg.at[pl.ds(off2, CH)], mbufs[sl],
                                         msems[sl])
                return carry

            lax.fori_loop(0, pairs, body, 0)

            toff = pl.multiple_of(base + full * CH, 8)
            pltpu.sync_copy(dst.at[pl.ds(toff, tail)], idx_t)
            pltpu.sync_copy(msg.at[pl.ds(toff, tail)], msg_t)
            pltpu.sync_copy(msg_t, accum.at[idx_t], add=True)

            plsc.subcore_barrier()

            @pl.when(s == 0)
            def _():
                pltpu.sync_copy(accum, out.at[comp * NC + c])

            plsc.subcore_barrier()

    return scatter


_NBLK = 2000


def _node_dense_body(nf_ref, w1_ref, b1_ref, w2_ref, b2_ref, out_ref):
    h = _silu(nf_ref[...] @ w1_ref[...] + b1_ref[...])
    out_ref[...] = h @ w2_ref[...] + b2_ref[...]


def _node_dense(nf, w1, b1, w2, b2):
    return pl.pallas_call(
        _node_dense_body,
        grid=(N // _NBLK,),
        in_specs=[
            pl.BlockSpec((_NBLK, F), lambda i: (i, 0)),
            pl.BlockSpec((F, F), lambda i: (0, 0)),
            pl.BlockSpec((F,), lambda i: (0,)),
            pl.BlockSpec((F, 3 * F), lambda i: (0, 0)),
            pl.BlockSpec((3 * F,), lambda i: (0,)),
        ],
        out_specs=pl.BlockSpec((_NBLK, 3 * F), lambda i: (i, 0)),
        out_shape=jax.ShapeDtypeStruct((N, 3 * F), jnp.float32),
    )(nf, w1, b1, w2, b2)


_EBLK = 2000


def _edge_body_common(diff, g_sc, fw_ref, fb_ref):
    d2 = jnp.sum(diff * diff, axis=1, keepdims=True)
    d = jnp.sqrt(d2 + 1e-12)
    k = lax.broadcasted_iota(jnp.int32, (diff.shape[0], NB), 1).astype(jnp.float32) + 1.0
    rbf = jnp.sin(d * k * (PI / CUTOFF)) / d
    fcut = jnp.where(d < CUTOFF, 0.5 * (jnp.cos(d * (PI / CUTOFF)) + 1.0), 0.0)
    filt = (rbf @ fw_ref[...] + fb_ref[...]) * fcut
    fo = filt * g_sc
    gate_v = fo[:, :F]
    gate_e = fo[:, F:2 * F]
    msg_s = fo[:, 2 * F:]
    unit = diff / d
    return gate_v, gate_e, msg_s, unit


def _edge_body(ed_ref, g_ref, fw_ref, fb_ref, ms_ref, mx_ref, my_ref, mz_ref):
    diff = ed_ref[...]
    g = g_ref[...]
    gate_v, gate_e, msg_s, unit = _edge_body_common(diff, g[:, :3 * F], fw_ref, fb_ref)
    ms_ref[...] = msg_s
    mx_ref[...] = g[:, 3 * F:4 * F] * gate_v + gate_e * unit[:, 0:1]
    my_ref[...] = g[:, 4 * F:5 * F] * gate_v + gate_e * unit[:, 1:2]
    mz_ref[...] = g[:, 5 * F:6 * F] * gate_v + gate_e * unit[:, 2:3]


def _edge_body_l0(ed_ref, g_ref, fw_ref, fb_ref, ms_ref, mx_ref, my_ref, mz_ref):
    diff = ed_ref[...]
    gate_v, gate_e, msg_s, unit = _edge_body_common(diff, g_ref[...], fw_ref, fb_ref)
    ms_ref[...] = msg_s
    mx_ref[...] = gate_e * unit[:, 0:1]
    my_ref[...] = gate_e * unit[:, 1:2]
    mz_ref[...] = gate_e * unit[:, 2:3]


def _edge_messages(edge_diff, gathered, fw, fb, first_layer):
    gdim = 3 * F if first_layer else 6 * F
    body = _edge_body_l0 if first_layer else _edge_body
    out_sd = jax.ShapeDtypeStruct((E, F), jnp.float32)
    out_spec = pl.BlockSpec((_EBLK, F), lambda i: (i, 0))
    return pl.pallas_call(
        body,
        grid=(E // _EBLK,),
        in_specs=[
            pl.BlockSpec((_EBLK, 3), lambda i: (i, 0)),
            pl.BlockSpec((_EBLK, gdim), lambda i: (i, 0)),
            pl.BlockSpec((NB, 3 * F), lambda i: (0, 0)),
            pl.BlockSpec((3 * F,), lambda i: (0,)),
        ],
        out_specs=[out_spec, out_spec, out_spec, out_spec],
        out_shape=[out_sd, out_sd, out_sd, out_sd],
    )(edge_diff, gathered, fw, fb)


def _update_body(nf_ref, nvx_ref, nvy_ref, nvz_ref, seg_ref,
                 u_ref, v_ref, w1_ref, b1_ref, w2_ref, b2_ref,
                 nf_out, nvx_out, nvy_out, nvz_out):
    seg = seg_ref[...]
    f = nf_ref[...] + seg[0] + seg[1]
    vx = nvx_ref[...] + seg[2] + seg[3]
    vy = nvy_ref[...] + seg[4] + seg[5]
    vz = nvz_ref[...] + seg[6] + seg[7]
    U = u_ref[...]
    V = v_ref[...]
    uvx = vx @ U
    uvy = vy @ U
    uvz = vz @ U
    vvx = vx @ V
    vvy = vy @ V
    vvz = vz @ V
    vn = jnp.sqrt(vvx * vvx + vvy * vvy + vvz * vvz + 1e-12)
    w1 = w1_ref[...]
    h = _silu(f @ w1[:F, :] + vn @ w1[F:, :] + b1_ref[...])
    a = h @ w2_ref[...] + b2_ref[...]
    a_vv = a[:, :F]
    a_sv = a[:, F:2 * F]
    a_ss = a[:, 2 * F:]
    nf_out[...] = f + (uvx * vvx + uvy * vvy + uvz * vvz) * a_sv + a_ss
    nvx_out[...] = vx + a_vv * uvx
    nvy_out[...] = vy + a_vv * uvy
    nvz_out[...] = vz + a_vv * uvz


_UBLK = 1000


def _update(nf, nv3, seg, U, V, w1, b1, w2, b2):
    nspec = pl.BlockSpec((_UBLK, F), lambda i: (i, 0))
    out_sd = jax.ShapeDtypeStruct((N, F), jnp.float32)
    seg_spec = pl.BlockSpec((4 * NC, _UBLK, F), lambda i: (0, i, 0))
    return pl.pallas_call(
        _update_body,
        grid=(N // _UBLK,),
        in_specs=[nspec, nspec, nspec, nspec, seg_spec] + [
            pl.BlockSpec((F, F), lambda i: (0, 0)),
            pl.BlockSpec((F, F), lambda i: (0, 0)),
            pl.BlockSpec((2 * F, F), lambda i: (0, 0)),
            pl.BlockSpec((F,), lambda i: (0,)),
            pl.BlockSpec((F, 3 * F), lambda i: (0, 0)),
            pl.BlockSpec((3 * F,), lambda i: (0,)),
        ],
        out_specs=[nspec, nspec, nspec, nspec],
        out_shape=[out_sd, out_sd, out_sd, out_sd],
    )(nf, nv3[0], nv3[1], nv3[2], seg, U, V, w1, b1, w2, b2)


def _readout_body(nf_ref, w1_ref, b1_ref, w2_ref, b2_ref, out_ref):
    h = _silu(nf_ref[...] @ w1_ref[...] + b1_ref[...])
    out_ref[...] = h @ w2_ref[...] + b2_ref[...]


def _readout(node_feat, ro_W1, ro_b1, ro_W2, ro_b2):
    out = pl.pallas_call(
        _readout_body,
        grid=(N // _NBLK,),
        in_specs=[
            pl.BlockSpec((_NBLK, F), lambda i: (i, 0)),
            pl.BlockSpec((F, F), lambda i: (0, 0)),
            pl.BlockSpec((F,), lambda i: (0,)),
            pl.BlockSpec((F, 1), lambda i: (0, 0)),
            pl.BlockSpec((1,), lambda i: (0,)),
        ],
        out_specs=pl.BlockSpec((_NBLK, 1), lambda i: (i, 0)),
        out_shape=jax.ShapeDtypeStruct((N, 1), jnp.float32),
    )(node_feat, ro_W1, ro_b1, ro_W2, ro_b2)
    return out[:, 0]


_N_PAD = 10240
_sc_cache = {}


def _embed_gather(table, idx):
    if "embed" not in _sc_cache:
        _sc_cache["embed"] = _make_sc_gather(119, F, _N_PAD)
    return _sc_cache["embed"](table, idx)


def _table_gather_l0(table, idx):
    if "g0" not in _sc_cache:
        _sc_cache["g0"] = _make_sc_gather(N, 3 * F, E)
    return _sc_cache["g0"](table, idx)


def _table_gather(table, idx):
    if "g" not in _sc_cache:
        _sc_cache["g"] = _make_sc_gather(N, 6 * F, E)
    return _sc_cache["g"](table, idx)


def _seg_scatter(*args):
    if "s" not in _sc_cache:
        _sc_cache["s"] = _make_sc_scatter(4)
    return _sc_cache["s"](*args)


def kernel(Z, edge_index, edge_diff, embed, msg_W1, msg_b1, msg_W2, msg_b2,
           filt_W, filt_b, upd_U, upd_V, upd_W1, upd_b1, upd_W2, upd_b2,
           ro_W1, ro_b1, ro_W2, ro_b2):
    src = edge_index[:, 1].astype(jnp.int32)
    dst = edge_index[:, 0].astype(jnp.int32)
    z_pad = jnp.zeros((_N_PAD,), jnp.int32).at[:N].set(Z.astype(jnp.int32))

    node_feat = _embed_gather(embed, z_pad)[:N]
    nv3 = None

    zeros_nf = jnp.zeros((N, F), jnp.float32)

    for l in range(3):
        sc = _node_dense(node_feat, msg_W1[l], msg_b1[l], msg_W2[l], msg_b2[l])
        if l == 0:
            gathered = _table_gather_l0(sc, src)
        else:
            table = jnp.concatenate([sc, nv3[0], nv3[1], nv3[2]], axis=1)
            gathered = _table_gather(table, src)
        ms, mx, my, mz = _edge_messages(edge_diff, gathered, filt_W[l],
                                        filt_b[l], l == 0)
        seg = _seg_scatter(ms, mx, my, mz, dst, zeros_nf)
        if l == 0:
            nv_in = (zeros_nf, zeros_nf, zeros_nf)
        else:
            nv_in = nv3
        node_feat, nvx, nvy, nvz = _update(
            node_feat, nv_in, seg, upd_U[l], upd_V[l],
            upd_W1[l], upd_b1[l], upd_W2[l], upd_b2[l])
        nv3 = (nvx, nvy, nvz)

    return _readout(node_feat, ro_W1, ro_b1, ro_W2, ro_b2)
